# baseline (speedup 1.0000x reference)
import functools

import jax
import jax.numpy as jnp
from jax import lax
from jax.experimental import pallas as pl
from jax.experimental.pallas import tpu as pltpu
from jax.experimental.pallas import tpu_sc as plsc

N = 10000
D = 128
E = 320000
NC = 2
NS = 16
NW = NC * NS
CHUNK = 128
NCH = 80
EP = NW * CHUNK * NCH
NDEG = 10240
NPAD = 10240
RSTRIPE = NPAD // NS
RB = 1000
GRID = N // RB


def _mesh():
    return plsc.VectorSubcoreMesh(core_axis_name="c", subcore_axis_name="s",
                                  num_cores=NC, num_subcores=NS)


@functools.partial(
    pl.kernel,
    out_type=(jax.ShapeDtypeStruct((NDEG,), jnp.float32),
              jax.ShapeDtypeStruct((NDEG,), jnp.float32)),
    mesh=_mesh(),
    scratch_types=[
        pltpu.VMEM((NCH, CHUNK), jnp.int32),
        pltpu.VMEM((NCH, CHUNK), jnp.float32),
        pltpu.VMEM((CHUNK,), jnp.float32),
        pltpu.VMEM_SHARED((NDEG,), jnp.float32),
    ],
)
def _deg_kernel(dst_hbm, w_hbm, out0, out1, didx_all, wv_all, zb, deg_sh):
    c = lax.axis_index("c")
    s = lax.axis_index("s")
    wid = c * NS + s

    pltpu.sync_copy(dst_hbm.at[wid], didx_all)
    pltpu.sync_copy(w_hbm.at[wid], wv_all)
    for j in range(8):
        zb[pl.ds(j * 16, 16)] = jnp.zeros((16,), jnp.float32)
    for k in range(5):
        pltpu.sync_copy(zb, deg_sh.at[pl.ds(s * 640 + k * CHUNK, CHUNK)])
    plsc.subcore_barrier()

    def body(g, carry):
        pltpu.sync_copy(wv_all.at[g], deg_sh.at[didx_all.at[g]], add=True)
        return carry

    lax.fori_loop(0, NCH, body, 0)
    plsc.subcore_barrier()

    @pl.when(c == 0)
    def _():
        pltpu.sync_copy(deg_sh.at[pl.ds(s * 640, 640)], out0.at[pl.ds(s * 640, 640)])

    @pl.when(c == 1)
    def _():
        pltpu.sync_copy(deg_sh.at[pl.ds(s * 640, 640)], out1.at[pl.ds(s * 640, 640)])


@functools.partial(
    pl.kernel,
    out_type=(jax.ShapeDtypeStruct((NPAD, D), jnp.float32),
              jax.ShapeDtypeStruct((NPAD, D), jnp.float32)),
    mesh=_mesh(),
    scratch_types=[
        pltpu.VMEM((CHUNK, D // 2), jnp.int32),
        pltpu.VMEM((CHUNK, D // 2), jnp.int32),
        pltpu.VMEM((CHUNK, D), jnp.float32),
        pltpu.VMEM((NCH // 2, CHUNK), jnp.int32),
        pltpu.VMEM((NCH // 2, CHUNK), jnp.int32),
        pltpu.VMEM((NCH // 2, CHUNK), jnp.float32),
        pltpu.VMEM_SHARED((NPAD, D), jnp.float32),
        pltpu.SemaphoreType.DMA,
        pltpu.SemaphoreType.DMA,
    ],
    compiler_params=pltpu.CompilerParams(use_tc_tiling_on_sc=False,
                                         needs_layout_passes=False),
)
def _scatter_kernel(hp_hbm, src_hbm, dst_hbm, w_hbm, out0, out1,
                    rows0, rows1, rowsf, sidx_all, didx_all, wv_all, s_sh,
                    sem0, sem1):
    c = lax.axis_index("c")
    s = lax.axis_index("s")
    wid = c * NS + s

    def zrow(r, carry):
        for j in range(8):
            rowsf[r, pl.ds(j * 16, 16)] = jnp.zeros((16,), jnp.float32)
        return carry

    lax.fori_loop(0, CHUNK, zrow, 0)
    base = s * RSTRIPE
    for k in range(RSTRIPE // CHUNK):
        pltpu.sync_copy(rowsf, s_sh.at[pl.ds(base + k * CHUNK, CHUNK)])
    plsc.subcore_barrier()

    def scale_scatter(g, rows):
        def sgrp(t, cc):
            wvec = wv_all[g, pl.ds(t * 16, 16)]
            for l in range(16):
                ws = wvec[l]
                r = t * 16 + l
                for j in range(4):
                    vi = rows[r, pl.ds(j * 16, 16)]
                    v32 = plsc.bitcast(vi, jnp.bfloat16)
                    a, b = plsc.unpack(v32, format=plsc.PackFormat.INTERLEAVED)
                    rowsf[r, pl.ds(j * 32, 16)] = a * ws
                    rowsf[r, pl.ds(j * 32 + 16, 16)] = b * ws
            return cc

        lax.fori_loop(0, CHUNK // 16, sgrp, 0)
        pltpu.sync_copy(rowsf, s_sh.at[didx_all.at[g]], add=True)

    def body(i, carry):
        g0 = 2 * i
        g1 = 2 * i + 1
        d0 = pltpu.async_copy(hp_hbm.at[sidx_all.at[g0]], rows0, sem0)
        d1 = pltpu.async_copy(hp_hbm.at[sidx_all.at[g1]], rows1, sem1)
        d0.wait()
        scale_scatter(g0, rows0)
        d1.wait()
        scale_scatter(g1, rows1)
        return carry

    nhalf = NCH // 2
    for h in range(2):
        pltpu.sync_copy(src_hbm.at[wid, pl.ds(h * nhalf, nhalf)], sidx_all)
        pltpu.sync_copy(dst_hbm.at[wid, pl.ds(h * nhalf, nhalf)], didx_all)
        pltpu.sync_copy(w_hbm.at[wid, pl.ds(h * nhalf, nhalf)], wv_all)
        lax.fori_loop(0, nhalf // 2, body, 0)
    plsc.subcore_barrier()

    @pl.when(c == 0)
    def _():
        pltpu.sync_copy(s_sh.at[pl.ds(s * RSTRIPE, RSTRIPE)],
                        out0.at[pl.ds(s * RSTRIPE, RSTRIPE)])

    @pl.when(c == 1)
    def _():
        pltpu.sync_copy(s_sh.at[pl.ds(s * RSTRIPE, RSTRIPE)],
                        out1.at[pl.ds(s * RSTRIPE, RSTRIPE)])


def _tc1_body(d1, d2, x_ref, w_ref, dinv_ref, h1p_ref):
    deg = d1[...] + d2[...] + 1.0
    dinv = jnp.where(deg > 0, lax.rsqrt(jnp.maximum(deg, 1e-12)), 0.0)
    dinv_ref[...] = dinv
    h = jnp.dot(x_ref[...], w_ref[...], preferred_element_type=jnp.float32)
    h1p_ref[...] = h * dinv


def _tc2_body(dinv_ref, h1p_ref, sa, sb, b1_ref, w2_ref, h2p_ref):
    dinv = dinv_ref[...]
    h1 = jnp.maximum(dinv * (sa[...] + sb[...] + h1p_ref[...]) + b1_ref[...], 0.0)
    h2p_ref[...] = jnp.dot(h1, w2_ref[...], preferred_element_type=jnp.float32) * dinv


def _tc3_body(dinv_ref, h2p_ref, sa, sb, b2_ref, wm_ref, bm_ref,
              out_ref, acc):
    i = pl.program_id(0)
    dinv = dinv_ref[...]
    h2 = jnp.maximum(dinv * (sa[...] + sb[...] + h2p_ref[...]) + b2_ref[...], 0.0)
    psum = jnp.sum(h2, axis=0, keepdims=True)

    @pl.when(i == 0)
    def _():
        acc[...] = psum

    @pl.when(i > 0)
    def _():
        acc[...] = acc[...] + psum

    @pl.when(i == GRID - 1)
    def _():
        out_ref[...] = (jnp.dot(acc[...] * (1.0 / N), wm_ref[...],
                                preferred_element_type=jnp.float32)
                        + bm_ref[...])


def _row_spec(width):
    return pl.BlockSpec((RB, width), lambda i: (i, 0))


def _full_spec(shape):
    return pl.BlockSpec(shape, lambda i: (0, 0))


_tc1 = pl.pallas_call(
    _tc1_body,
    grid=(GRID,),
    in_specs=[_row_spec(1), _row_spec(1), _row_spec(D), _full_spec((D, D))],
    out_specs=[_row_spec(1), _row_spec(D)],
    out_shape=[jax.ShapeDtypeStruct((N, 1), jnp.float32),
               jax.ShapeDtypeStruct((N, D), jnp.float32)],
)

_tc2 = pl.pallas_call(
    _tc2_body,
    grid=(GRID,),
    in_specs=[_row_spec(1), _row_spec(D), _row_spec(D), _row_spec(D),
              _full_spec((1, D)), _full_spec((D, D))],
    out_specs=pl.BlockSpec((RB, D), lambda i: (i, 0)),
    out_shape=jax.ShapeDtypeStruct((N, D), jnp.float32),
)


def _make_tc3(nclass):
    return pl.pallas_call(
        _tc3_body,
        grid=(GRID,),
        in_specs=[_row_spec(1), _row_spec(D), _row_spec(D), _row_spec(D),
                  _full_spec((1, D)), _full_spec((D, nclass)),
                  _full_spec((1, nclass))],
        out_specs=pl.BlockSpec((1, nclass), lambda i: (0, 0)),
        out_shape=jax.ShapeDtypeStruct((1, nclass), jnp.float32),
        scratch_shapes=[pltpu.VMEM((1, D), jnp.float32)],
    )


def kernel(x, edge_index, edge_attr, W1, b1, W2, b2, Wm, bm):
    src = edge_index[0].astype(jnp.int32)
    dst = edge_index[1].astype(jnp.int32)
    w = edge_attr.astype(jnp.float32)
    pad = EP - E
    srcp = jnp.concatenate([src, jnp.zeros((pad,), jnp.int32)]).reshape(NW, NCH, CHUNK)
    dstp = jnp.concatenate([dst, jnp.zeros((pad,), jnp.int32)]).reshape(NW, NCH, CHUNK)
    wp = jnp.concatenate([w, jnp.zeros((pad,), jnp.float32)]).reshape(NW, NCH, CHUNK)

    dega, degb = _deg_kernel(dstp, wp)
    deg1 = dega[:N].reshape(N, 1)
    deg2 = degb[:N].reshape(N, 1)

    def _bf16_interleaved(h):
        hb = h.astype(jnp.bfloat16)
        hb = hb.reshape(N, D // 32, 2, 16).transpose(0, 1, 3, 2)
        return jax.lax.bitcast_convert_type(
            hb.reshape(N, D // 2, 2), jnp.int32)

    dinv, h1p = _tc1(deg1, deg2, x, W1)
    s1a, s1b = _scatter_kernel(_bf16_interleaved(h1p), srcp, dstp, wp)
    h2p = _tc2(dinv, h1p, s1a[:N], s1b[:N], b1.reshape(1, D), W2)
    s2a, s2b = _scatter_kernel(_bf16_interleaved(h2p), srcp, dstp, wp)
    nclass = Wm.shape[1]
    out = _make_tc3(nclass)(dinv, h2p, s2a[:N], s2b[:N], b2.reshape(1, D),
                            Wm, bm.reshape(1, nclass))
    return out.reshape(nclass)


# speedup vs baseline: 1.2981x; 1.2981x over previous
import functools

import jax
import jax.numpy as jnp
from jax import lax
from jax.experimental import pallas as pl
from jax.experimental.pallas import tpu as pltpu
from jax.experimental.pallas import tpu_sc as plsc

N = 10000
D = 128
E = 320000
NC = 2
NS = 16
NW = NC * NS
CHUNK = 128
NCH = 80
EP = NW * CHUNK * NCH
NDEG = 10240
NPAD = 10240
RSTRIPE = NPAD // NS
RB = 1000
GRID = N // RB


def _mesh():
    return plsc.VectorSubcoreMesh(core_axis_name="c", subcore_axis_name="s",
                                  num_cores=NC, num_subcores=NS)


@functools.partial(
    pl.kernel,
    out_type=(jax.ShapeDtypeStruct((NDEG,), jnp.float32),
              jax.ShapeDtypeStruct((NDEG,), jnp.float32)),
    mesh=_mesh(),
    scratch_types=[
        pltpu.VMEM((NCH, CHUNK), jnp.int32),
        pltpu.VMEM((NCH, CHUNK), jnp.float32),
        pltpu.VMEM((CHUNK,), jnp.float32),
        pltpu.VMEM_SHARED((NDEG,), jnp.float32),
    ],
)
def _deg_kernel(dst_hbm, w_hbm, out0, out1, didx_all, wv_all, zb, deg_sh):
    c = lax.axis_index("c")
    s = lax.axis_index("s")
    wid = c * NS + s

    pltpu.sync_copy(dst_hbm.at[wid], didx_all)
    pltpu.sync_copy(w_hbm.at[wid], wv_all)
    for j in range(8):
        zb[pl.ds(j * 16, 16)] = jnp.zeros((16,), jnp.float32)
    for k in range(5):
        pltpu.sync_copy(zb, deg_sh.at[pl.ds(s * 640 + k * CHUNK, CHUNK)])
    plsc.subcore_barrier()

    def body(g, carry):
        pltpu.sync_copy(wv_all.at[g], deg_sh.at[didx_all.at[g]], add=True)
        return carry

    lax.fori_loop(0, NCH, body, 0)
    plsc.subcore_barrier()

    @pl.when(c == 0)
    def _():
        pltpu.sync_copy(deg_sh.at[pl.ds(s * 640, 640)], out0.at[pl.ds(s * 640, 640)])

    @pl.when(c == 1)
    def _():
        pltpu.sync_copy(deg_sh.at[pl.ds(s * 640, 640)], out1.at[pl.ds(s * 640, 640)])


@functools.partial(
    pl.kernel,
    out_type=(jax.ShapeDtypeStruct((NPAD, D), jnp.float32),
              jax.ShapeDtypeStruct((NPAD, D), jnp.float32)),
    mesh=_mesh(),
    scratch_types=[
        pltpu.VMEM((CHUNK, D // 2), jnp.int32),
        pltpu.VMEM((CHUNK, D // 2), jnp.int32),
        pltpu.VMEM((CHUNK, D), jnp.float32),
        pltpu.VMEM((NCH // 2, CHUNK), jnp.int32),
        pltpu.VMEM((NCH // 2, CHUNK), jnp.int32),
        pltpu.VMEM((NCH // 2, CHUNK), jnp.float32),
        pltpu.VMEM_SHARED((NPAD, D), jnp.float32),
        pltpu.SemaphoreType.DMA,
        pltpu.SemaphoreType.DMA,
    ],
    compiler_params=pltpu.CompilerParams(use_tc_tiling_on_sc=False,
                                         needs_layout_passes=False),
)
def _scatter_kernel(hp_hbm, src_hbm, dst_hbm, w_hbm, out0, out1,
                    rows0, rows1, rowsf, sidx_all, didx_all, wv_all, s_sh,
                    sem0, sem1):
    c = lax.axis_index("c")
    s = lax.axis_index("s")
    wid = c * NS + s

    def zrow(r, carry):
        for j in range(8):
            rowsf[r, pl.ds(j * 16, 16)] = jnp.zeros((16,), jnp.float32)
        return carry

    lax.fori_loop(0, CHUNK, zrow, 0)
    base = s * RSTRIPE
    for k in range(RSTRIPE // CHUNK):
        pltpu.sync_copy(rowsf, s_sh.at[pl.ds(base + k * CHUNK, CHUNK)])
    plsc.subcore_barrier()

    def scale_scatter(g, rows):
        for t in range(CHUNK // 16):
            wvec = wv_all[g, pl.ds(t * 16, 16)]
            for l in range(16):
                ws = wvec[l]
                r = t * 16 + l
                for j in range(4):
                    vi = rows[r, pl.ds(j * 16, 16)]
                    a = plsc.bitcast(vi << 16, jnp.float32)
                    b = plsc.bitcast(vi, jnp.float32)
                    rowsf[r, pl.ds(j * 32, 16)] = a * ws
                    rowsf[r, pl.ds(j * 32 + 16, 16)] = b * ws

        pltpu.sync_copy(rowsf, s_sh.at[didx_all.at[g]], add=True)

    def body(i, carry):
        g0 = 2 * i
        g1 = 2 * i + 1
        d0 = pltpu.async_copy(hp_hbm.at[sidx_all.at[g0]], rows0, sem0)
        d1 = pltpu.async_copy(hp_hbm.at[sidx_all.at[g1]], rows1, sem1)
        d0.wait()
        scale_scatter(g0, rows0)
        d1.wait()
        scale_scatter(g1, rows1)
        return carry

    nhalf = NCH // 2
    for h in range(2):
        pltpu.sync_copy(src_hbm.at[wid, pl.ds(h * nhalf, nhalf)], sidx_all)
        pltpu.sync_copy(dst_hbm.at[wid, pl.ds(h * nhalf, nhalf)], didx_all)
        pltpu.sync_copy(w_hbm.at[wid, pl.ds(h * nhalf, nhalf)], wv_all)
        lax.fori_loop(0, nhalf // 2, body, 0)
    plsc.subcore_barrier()

    @pl.when(c == 0)
    def _():
        pltpu.sync_copy(s_sh.at[pl.ds(s * RSTRIPE, RSTRIPE)],
                        out0.at[pl.ds(s * RSTRIPE, RSTRIPE)])

    @pl.when(c == 1)
    def _():
        pltpu.sync_copy(s_sh.at[pl.ds(s * RSTRIPE, RSTRIPE)],
                        out1.at[pl.ds(s * RSTRIPE, RSTRIPE)])


def _tc1_body(d1, d2, x_ref, w_ref, dinv_ref, h1p_ref):
    deg = d1[...] + d2[...] + 1.0
    dinv = jnp.where(deg > 0, lax.rsqrt(jnp.maximum(deg, 1e-12)), 0.0)
    dinv_ref[...] = dinv
    h = jnp.dot(x_ref[...], w_ref[...], preferred_element_type=jnp.float32)
    h1p_ref[...] = h * dinv


def _tc2_body(dinv_ref, h1p_ref, sa, sb, b1_ref, w2_ref, h2p_ref):
    dinv = dinv_ref[...]
    h1 = jnp.maximum(dinv * (sa[...] + sb[...] + h1p_ref[...]) + b1_ref[...], 0.0)
    h2p_ref[...] = jnp.dot(h1, w2_ref[...], preferred_element_type=jnp.float32) * dinv


def _tc3_body(dinv_ref, h2p_ref, sa, sb, b2_ref, wm_ref, bm_ref,
              out_ref, acc):
    i = pl.program_id(0)
    dinv = dinv_ref[...]
    h2 = jnp.maximum(dinv * (sa[...] + sb[...] + h2p_ref[...]) + b2_ref[...], 0.0)
    psum = jnp.sum(h2, axis=0, keepdims=True)

    @pl.when(i == 0)
    def _():
        acc[...] = psum

    @pl.when(i > 0)
    def _():
        acc[...] = acc[...] + psum

    @pl.when(i == GRID - 1)
    def _():
        out_ref[...] = (jnp.dot(acc[...] * (1.0 / N), wm_ref[...],
                                preferred_element_type=jnp.float32)
                        + bm_ref[...])


def _row_spec(width):
    return pl.BlockSpec((RB, width), lambda i: (i, 0))


def _full_spec(shape):
    return pl.BlockSpec(shape, lambda i: (0, 0))


_tc1 = pl.pallas_call(
    _tc1_body,
    grid=(GRID,),
    in_specs=[_row_spec(1), _row_spec(1), _row_spec(D), _full_spec((D, D))],
    out_specs=[_row_spec(1), _row_spec(D)],
    out_shape=[jax.ShapeDtypeStruct((N, 1), jnp.float32),
               jax.ShapeDtypeStruct((N, D), jnp.float32)],
)

_tc2 = pl.pallas_call(
    _tc2_body,
    grid=(GRID,),
    in_specs=[_row_spec(1), _row_spec(D), _row_spec(D), _row_spec(D),
              _full_spec((1, D)), _full_spec((D, D))],
    out_specs=pl.BlockSpec((RB, D), lambda i: (i, 0)),
    out_shape=jax.ShapeDtypeStruct((N, D), jnp.float32),
)


def _make_tc3(nclass):
    return pl.pallas_call(
        _tc3_body,
        grid=(GRID,),
        in_specs=[_row_spec(1), _row_spec(D), _row_spec(D), _row_spec(D),
                  _full_spec((1, D)), _full_spec((D, nclass)),
                  _full_spec((1, nclass))],
        out_specs=pl.BlockSpec((1, nclass), lambda i: (0, 0)),
        out_shape=jax.ShapeDtypeStruct((1, nclass), jnp.float32),
        scratch_shapes=[pltpu.VMEM((1, D), jnp.float32)],
    )


def kernel(x, edge_index, edge_attr, W1, b1, W2, b2, Wm, bm):
    src = edge_index[0].astype(jnp.int32)
    dst = edge_index[1].astype(jnp.int32)
    w = edge_attr.astype(jnp.float32)
    pad = EP - E
    srcp = jnp.concatenate([src, jnp.zeros((pad,), jnp.int32)]).reshape(NW, NCH, CHUNK)
    dstp = jnp.concatenate([dst, jnp.zeros((pad,), jnp.int32)]).reshape(NW, NCH, CHUNK)
    wp = jnp.concatenate([w, jnp.zeros((pad,), jnp.float32)]).reshape(NW, NCH, CHUNK)

    dega, degb = _deg_kernel(dstp, wp)
    deg1 = dega[:N].reshape(N, 1)
    deg2 = degb[:N].reshape(N, 1)

    def _bf16_interleaved(h):
        hb = h.astype(jnp.bfloat16)
        hb = hb.reshape(N, D // 32, 2, 16).transpose(0, 1, 3, 2)
        return jax.lax.bitcast_convert_type(
            hb.reshape(N, D // 2, 2), jnp.int32)

    dinv, h1p = _tc1(deg1, deg2, x, W1)
    s1a, s1b = _scatter_kernel(_bf16_interleaved(h1p), srcp, dstp, wp)
    h2p = _tc2(dinv, h1p, s1a[:N], s1b[:N], b1.reshape(1, D), W2)
    s2a, s2b = _scatter_kernel(_bf16_interleaved(h2p), srcp, dstp, wp)
    nclass = Wm.shape[1]
    out = _make_tc3(nclass)(dinv, h2p, s2a[:N], s2b[:N], b2.reshape(1, D),
                            Wm, bm.reshape(1, nclass))
    return out.reshape(nclass)


# speedup vs baseline: 1.6833x; 1.2967x over previous
import functools

import jax
import jax.numpy as jnp
from jax import lax
from jax.experimental import pallas as pl
from jax.experimental.pallas import tpu as pltpu
from jax.experimental.pallas import tpu_sc as plsc

N = 10000
D = 128
E = 320000
NC = 2
NS = 16
NW = NC * NS
CHUNK = 128
NCH = 80
EP = NW * CHUNK * NCH
NDEG = 10240
NPAD = 10240
RSTRIPE = NPAD // NS
RB = 1000
GRID = N // RB


def _mesh():
    return plsc.VectorSubcoreMesh(core_axis_name="c", subcore_axis_name="s",
                                  num_cores=NC, num_subcores=NS)


@functools.partial(
    pl.kernel,
    out_type=(jax.ShapeDtypeStruct((NDEG,), jnp.float32),
              jax.ShapeDtypeStruct((NDEG,), jnp.float32)),
    mesh=_mesh(),
    scratch_types=[
        pltpu.VMEM((NCH, CHUNK), jnp.int32),
        pltpu.VMEM((NCH, CHUNK), jnp.float32),
        pltpu.VMEM((CHUNK,), jnp.float32),
        pltpu.VMEM_SHARED((NDEG,), jnp.float32),
    ],
)
def _deg_kernel(dst_hbm, w_hbm, out0, out1, didx_all, wv_all, zb, deg_sh):
    c = lax.axis_index("c")
    s = lax.axis_index("s")
    wid = c * NS + s

    pltpu.sync_copy(dst_hbm.at[wid], didx_all)
    pltpu.sync_copy(w_hbm.at[wid], wv_all)
    for j in range(8):
        zb[pl.ds(j * 16, 16)] = jnp.zeros((16,), jnp.float32)
    for k in range(5):
        pltpu.sync_copy(zb, deg_sh.at[pl.ds(s * 640 + k * CHUNK, CHUNK)])
    plsc.subcore_barrier()

    def body(g, carry):
        pltpu.sync_copy(wv_all.at[g], deg_sh.at[didx_all.at[g]], add=True)
        return carry

    lax.fori_loop(0, NCH, body, 0)
    plsc.subcore_barrier()

    @pl.when(c == 0)
    def _():
        pltpu.sync_copy(deg_sh.at[pl.ds(s * 640, 640)], out0.at[pl.ds(s * 640, 640)])

    @pl.when(c == 1)
    def _():
        pltpu.sync_copy(deg_sh.at[pl.ds(s * 640, 640)], out1.at[pl.ds(s * 640, 640)])


@functools.partial(
    pl.kernel,
    out_type=(jax.ShapeDtypeStruct((NPAD, D), jnp.float32),
              jax.ShapeDtypeStruct((NPAD, D), jnp.float32)),
    mesh=_mesh(),
    scratch_types=[
        pltpu.VMEM((CHUNK, D // 2), jnp.int32),
        pltpu.VMEM((CHUNK, D // 2), jnp.int32),
        pltpu.VMEM((CHUNK, D), jnp.float32),
        pltpu.VMEM((CHUNK,), jnp.int32),
        pltpu.VMEM((CHUNK,), jnp.int32),
        pltpu.VMEM((NCH // 2, CHUNK), jnp.int32),
        pltpu.VMEM((NCH // 2, CHUNK), jnp.float32),
        pltpu.VMEM_SHARED((NPAD, D), jnp.float32),
        pltpu.SemaphoreType.DMA,
        pltpu.SemaphoreType.DMA,
    ],
    compiler_params=pltpu.CompilerParams(use_tc_tiling_on_sc=False,
                                         needs_layout_passes=False),
)
def _scatter_kernel(hp_hbm, src_hbm, dst_hbm, w_hbm, out0, out1,
                    rows0, rows1, rowsf, sidx0, sidx1,
                    didx_all, wv_all, s_sh, sem0, sem1):
    c = lax.axis_index("c")
    s = lax.axis_index("s")
    wid = c * NS + s

    def zrow(r, carry):
        for j in range(8):
            rowsf[r, pl.ds(j * 16, 16)] = jnp.zeros((16,), jnp.float32)
        return carry

    lax.fori_loop(0, CHUNK, zrow, 0)
    base = s * RSTRIPE
    for k in range(RSTRIPE // CHUNK):
        pltpu.sync_copy(rowsf, s_sh.at[pl.ds(base + k * CHUNK, CHUNK)])
    plsc.subcore_barrier()

    def scale_scatter(g, rows):
        for t in range(CHUNK // 16):
            wvec = wv_all[g, pl.ds(t * 16, 16)]
            for l in range(16):
                ws = wvec[l]
                r = t * 16 + l
                for j in range(4):
                    vi = rows[r, pl.ds(j * 16, 16)]
                    a = plsc.bitcast(vi << 16, jnp.float32)
                    b = plsc.bitcast(vi, jnp.float32)
                    rowsf[r, pl.ds(j * 32, 16)] = a * ws
                    rowsf[r, pl.ds(j * 32 + 16, 16)] = b * ws

        pltpu.sync_copy(rowsf, s_sh.at[didx_all.at[g]], add=True)

    bufs = ((rows0, sidx0, sem0), (rows1, sidx1, sem1))

    def issue(g, b):
        rows, sidx, sem = bufs[b]

        def _go():
            pltpu.sync_copy(src_hbm.at[wid, g], sidx)
            pltpu.async_copy(hp_hbm.at[sidx], rows, sem)

        if isinstance(g, int):
            _go()
        else:
            pl.when(g < NCH)(_go)

    def wait_process(gg, b):
        rows, sidx, sem = bufs[b]
        pltpu.make_async_copy(hp_hbm.at[sidx], rows, sem).wait()
        scale_scatter(gg, rows)

    issue(0, 0)
    issue(1, 1)
    nhalf = NCH // 2
    for h in range(2):
        pltpu.sync_copy(dst_hbm.at[wid, pl.ds(h * nhalf, nhalf)], didx_all)
        pltpu.sync_copy(w_hbm.at[wid, pl.ds(h * nhalf, nhalf)], wv_all)

        def body(i, carry):
            g = h * nhalf + 2 * i
            wait_process(2 * i, 0)
            issue(g + 2, 0)
            wait_process(2 * i + 1, 1)
            issue(g + 3, 1)
            return carry

        lax.fori_loop(0, nhalf // 2, body, 0)
    plsc.subcore_barrier()

    @pl.when(c == 0)
    def _():
        pltpu.sync_copy(s_sh.at[pl.ds(s * RSTRIPE, RSTRIPE)],
                        out0.at[pl.ds(s * RSTRIPE, RSTRIPE)])

    @pl.when(c == 1)
    def _():
        pltpu.sync_copy(s_sh.at[pl.ds(s * RSTRIPE, RSTRIPE)],
                        out1.at[pl.ds(s * RSTRIPE, RSTRIPE)])


def _tc1_body(d1, d2, x_ref, w_ref, dinv_ref, h1p_ref):
    deg = d1[...] + d2[...] + 1.0
    dinv = jnp.where(deg > 0, lax.rsqrt(jnp.maximum(deg, 1e-12)), 0.0)
    dinv_ref[...] = dinv
    h = jnp.dot(x_ref[...], w_ref[...], preferred_element_type=jnp.float32)
    h1p_ref[...] = h * dinv


def _tc2_body(dinv_ref, h1p_ref, sa, sb, b1_ref, w2_ref, h2p_ref):
    dinv = dinv_ref[...]
    h1 = jnp.maximum(dinv * (sa[...] + sb[...] + h1p_ref[...]) + b1_ref[...], 0.0)
    h2p_ref[...] = jnp.dot(h1, w2_ref[...], preferred_element_type=jnp.float32) * dinv


def _tc3_body(dinv_ref, h2p_ref, sa, sb, b2_ref, wm_ref, bm_ref,
              out_ref, acc):
    i = pl.program_id(0)
    dinv = dinv_ref[...]
    h2 = jnp.maximum(dinv * (sa[...] + sb[...] + h2p_ref[...]) + b2_ref[...], 0.0)
    psum = jnp.sum(h2, axis=0, keepdims=True)

    @pl.when(i == 0)
    def _():
        acc[...] = psum

    @pl.when(i > 0)
    def _():
        acc[...] = acc[...] + psum

    @pl.when(i == GRID - 1)
    def _():
        out_ref[...] = (jnp.dot(acc[...] * (1.0 / N), wm_ref[...],
                                preferred_element_type=jnp.float32)
                        + bm_ref[...])


def _row_spec(width):
    return pl.BlockSpec((RB, width), lambda i: (i, 0))


def _full_spec(shape):
    return pl.BlockSpec(shape, lambda i: (0, 0))


_tc1 = pl.pallas_call(
    _tc1_body,
    grid=(GRID,),
    in_specs=[_row_spec(1), _row_spec(1), _row_spec(D), _full_spec((D, D))],
    out_specs=[_row_spec(1), _row_spec(D)],
    out_shape=[jax.ShapeDtypeStruct((N, 1), jnp.float32),
               jax.ShapeDtypeStruct((N, D), jnp.float32)],
)

_tc2 = pl.pallas_call(
    _tc2_body,
    grid=(GRID,),
    in_specs=[_row_spec(1), _row_spec(D), _row_spec(D), _row_spec(D),
              _full_spec((1, D)), _full_spec((D, D))],
    out_specs=pl.BlockSpec((RB, D), lambda i: (i, 0)),
    out_shape=jax.ShapeDtypeStruct((N, D), jnp.float32),
)


def _make_tc3(nclass):
    return pl.pallas_call(
        _tc3_body,
        grid=(GRID,),
        in_specs=[_row_spec(1), _row_spec(D), _row_spec(D), _row_spec(D),
                  _full_spec((1, D)), _full_spec((D, nclass)),
                  _full_spec((1, nclass))],
        out_specs=pl.BlockSpec((1, nclass), lambda i: (0, 0)),
        out_shape=jax.ShapeDtypeStruct((1, nclass), jnp.float32),
        scratch_shapes=[pltpu.VMEM((1, D), jnp.float32)],
    )


def kernel(x, edge_index, edge_attr, W1, b1, W2, b2, Wm, bm):
    src = edge_index[0].astype(jnp.int32)
    dst = edge_index[1].astype(jnp.int32)
    w = edge_attr.astype(jnp.float32)
    pad = EP - E
    srcp = jnp.concatenate([src, jnp.zeros((pad,), jnp.int32)]).reshape(NW, NCH, CHUNK)
    dstp = jnp.concatenate([dst, jnp.zeros((pad,), jnp.int32)]).reshape(NW, NCH, CHUNK)
    wp = jnp.concatenate([w, jnp.zeros((pad,), jnp.float32)]).reshape(NW, NCH, CHUNK)

    dega, degb = _deg_kernel(dstp, wp)
    deg1 = dega[:N].reshape(N, 1)
    deg2 = degb[:N].reshape(N, 1)

    def _bf16_interleaved(h):
        hb = h.astype(jnp.bfloat16)
        hb = hb.reshape(N, D // 32, 2, 16).transpose(0, 1, 3, 2)
        return jax.lax.bitcast_convert_type(
            hb.reshape(N, D // 2, 2), jnp.int32)

    dinv, h1p = _tc1(deg1, deg2, x, W1)
    s1a, s1b = _scatter_kernel(_bf16_interleaved(h1p), srcp, dstp, wp)
    h2p = _tc2(dinv, h1p, s1a[:N], s1b[:N], b1.reshape(1, D), W2)
    s2a, s2b = _scatter_kernel(_bf16_interleaved(h2p), srcp, dstp, wp)
    nclass = Wm.shape[1]
    out = _make_tc3(nclass)(dinv, h2p, s2a[:N], s2b[:N], b2.reshape(1, D),
                            Wm, bm.reshape(1, nclass))
    return out.reshape(nclass)
